# baseline (speedup 1.0000x reference)
import jax
import jax.numpy as jnp
from jax.experimental import pallas as pl
from jax.experimental.pallas import tpu as pltpu


def _batchnorm(x, g, b, eps=1e-06):
    m = x.mean(0)
    v = x.var(0)
    return g * (x - m) / jnp.sqrt(v + eps) + b


def _layernorm(x, g, b, eps=1e-05):
    m = x.mean(-1, keepdims=True)
    v = x.var(-1, keepdims=True)
    return g * (x - m) / jnp.sqrt(v + eps) + b


def _gcn(x, ei, W, b):
    n = x.shape[0]
    sl = jnp.arange(n)
    row = jnp.concatenate([ei[0], sl])
    col = jnp.concatenate([ei[1], sl])
    deg = jnp.zeros((n,), x.dtype).at[col].add(1.0)
    dis = jax.lax.rsqrt(deg)
    norm = dis[row] * dis[col]
    xw = x @ W
    out = jnp.zeros_like(xw).at[col].add(xw[row] * norm[:, None])
    return out + b


def _gat(x, ei, W, a_s, a_d, b, H=4, C=256):
    n = x.shape[0]
    xw = (x @ W).reshape(n, H, C)
    asrc = (xw * a_s).sum(-1)
    adst = (xw * a_d).sum(-1)
    sl = jnp.arange(n)
    row = jnp.concatenate([ei[0], sl])
    col = jnp.concatenate([ei[1], sl])
    al = jax.nn.leaky_relu(asrc[row] + adst[col], 0.2)
    mx = jax.ops.segment_max(al, col, num_segments=n)
    al = jnp.exp(al - mx[col])
    sm = jax.ops.segment_sum(al, col, num_segments=n)
    al = al / (sm[col] + 1e-16)
    out = jax.ops.segment_sum(xw[row] * al[:, :, None], col, num_segments=n)
    return out.reshape(n, H * C) + b


def _heads_body(h_ref, w_ref, b_ref, o_ref):
    acc = jnp.dot(h_ref[...], w_ref[...], preferred_element_type=jnp.float32)
    o_ref[...] = jax.nn.leaky_relu(acc + b_ref[...], 0.01)


def _heads(h, w_cat, b_cat):
    e, c = h.shape
    blk = 2000
    return pl.pallas_call(
        _heads_body,
        grid=(e // blk,),
        in_specs=[
            pl.BlockSpec((blk, c), lambda i: (i, 0)),
            pl.BlockSpec((c, 128), lambda i: (0, 0)),
            pl.BlockSpec((1, 128), lambda i: (0, 0)),
        ],
        out_specs=pl.BlockSpec((blk, 128), lambda i: (i, 0)),
        out_shape=jax.ShapeDtypeStruct((e, 128), jnp.float32),
    )(h, w_cat, b_cat)


def kernel(x, edge_index, pos, y, emb, fc3_w, fc3_b, W1, b1, W2, b2, Wg, att_src, att_dst, bg, W3, b3, fc1_w, fc1_b, fc2_w, fc2_b, bn1_g, bn1_b, bn2_g, bn2_b, bn3_g, bn3_b, bn4_g, bn4_b, bn5_g, bn5_b, ln1_g, ln1_b, ln2_g, ln2_b):
    ei = edge_index - 1
    row, col = ei[0], ei[1]
    ef = (pos[col] - pos[row]).reshape(-1, 3)
    ef = _batchnorm(ef @ fc3_w + fc3_b, bn4_g, bn4_b)
    h = _batchnorm(emb[x], bn5_g, bn5_b)
    h = jnp.concatenate([h[row], h[col]], axis=1)
    h = jnp.concatenate([h, ef], axis=1)
    h = jax.nn.leaky_relu(_gcn(h, y, W1, b1), 0.01)
    h = _batchnorm(h, bn1_g, bn1_b)
    h = _layernorm(h, ln1_g, ln1_b)
    h = jax.nn.leaky_relu(h + _gcn(h, y, W2, b2), 0.01)
    h = _batchnorm(h, bn2_g, bn2_b)
    h = _gat(h, y, Wg, att_src, att_dst, bg)
    h = _layernorm(h, ln2_g, ln2_b)
    h = jax.nn.leaky_relu(_gcn(h, y, W3, b3), 0.01)
    h = _batchnorm(h, bn3_g, bn3_b)
    w_cat = jnp.concatenate(
        [fc1_w, fc2_w, jnp.zeros((256, 111), jnp.float32)], axis=1)
    b_cat = jnp.concatenate(
        [fc1_b, fc2_b, jnp.zeros((111,), jnp.float32)]).reshape(1, 128)
    out = _heads(h, w_cat, b_cat)
    sym = out[:, 0:1]
    asym = out[:, 1:17]
    return (sym, asym)


# speedup vs baseline: 1.2118x; 1.2118x over previous
import functools

import jax
import jax.numpy as jnp
from jax.experimental import pallas as pl
from jax.experimental.pallas import tpu as pltpu

_BLK = 2000


def _batchnorm(x, g, b, eps=1e-06):
    m = x.mean(0)
    v = x.var(0)
    return g * (x - m) / jnp.sqrt(v + eps) + b


def _layernorm(x, g, b, eps=1e-05):
    m = x.mean(-1, keepdims=True)
    v = x.var(-1, keepdims=True)
    return g * (x - m) / jnp.sqrt(v + eps) + b


def _mm_body(x_ref, w_ref, b_ref, o_ref, *, act):
    acc = jnp.dot(x_ref[...], w_ref[...], preferred_element_type=jnp.float32)
    acc = acc + b_ref[...]
    if act:
        acc = jax.nn.leaky_relu(acc, 0.01)
    o_ref[...] = acc


def _mm(x, w, b, act=False):
    e, k = x.shape
    c = w.shape[1]
    return pl.pallas_call(
        functools.partial(_mm_body, act=act),
        grid=(e // _BLK,),
        in_specs=[
            pl.BlockSpec((_BLK, k), lambda i: (i, 0)),
            pl.BlockSpec((k, c), lambda i: (0, 0)),
            pl.BlockSpec((1, c), lambda i: (0, 0)),
        ],
        out_specs=pl.BlockSpec((_BLK, c), lambda i: (i, 0)),
        out_shape=jax.ShapeDtypeStruct((e, c), jnp.float32),
    )(x, w, b.reshape(1, c))


def _gcn(x, row, col, dis, W, b):
    xw = _mm(x, W, jnp.zeros((W.shape[1],), jnp.float32))
    norm = dis[row] * dis[col]
    out = jnp.zeros_like(xw).at[col].add(xw[row] * norm[:, None])
    return out + xw * (dis * dis)[:, None] + b


def _gat(x, row, col, W, a_s, a_d, b, H=4, C=256):
    n = x.shape[0]
    xw = _mm(x, W, jnp.zeros((W.shape[1],), jnp.float32)).reshape(n, H, C)
    asrc = (xw * a_s).sum(-1)
    adst = (xw * a_d).sum(-1)
    al_self = jax.nn.leaky_relu(asrc + adst, 0.2)
    al_e = jax.nn.leaky_relu(asrc[row] + adst[col], 0.2)
    mx = jnp.maximum(jax.ops.segment_max(al_e, col, num_segments=n), al_self)
    al_e = jnp.exp(al_e - mx[col])
    w_self = jnp.exp(al_self - mx)
    sm = jax.ops.segment_sum(al_e, col, num_segments=n) + w_self + 1e-16
    al_e = al_e / sm[col]
    out = jax.ops.segment_sum(xw[row] * al_e[:, :, None], col, num_segments=n)
    out = out + xw * (w_self / sm)[:, :, None]
    return out.reshape(n, H * C) + b


def kernel(x, edge_index, pos, y, emb, fc3_w, fc3_b, W1, b1, W2, b2, Wg, att_src, att_dst, bg, W3, b3, fc1_w, fc1_b, fc2_w, fc2_b, bn1_g, bn1_b, bn2_g, bn2_b, bn3_g, bn3_b, bn4_g, bn4_b, bn5_g, bn5_b, ln1_g, ln1_b, ln2_g, ln2_b):
    ei = edge_index - 1
    row0, col0 = ei[0], ei[1]
    ef = (pos[col0] - pos[row0]).reshape(-1, 3)
    ef = _batchnorm(ef @ fc3_w + fc3_b, bn4_g, bn4_b)
    h = _batchnorm(emb[x], bn5_g, bn5_b)
    h = jnp.concatenate([h[row0], h[col0], ef], axis=1)

    e = h.shape[0]
    row, col = y[0], y[1]
    deg = jnp.ones((e,), jnp.float32).at[col].add(1.0)
    dis = jax.lax.rsqrt(deg)

    h = jax.nn.leaky_relu(_gcn(h, row, col, dis, W1, b1), 0.01)
    h = _batchnorm(h, bn1_g, bn1_b)
    h = _layernorm(h, ln1_g, ln1_b)
    h = jax.nn.leaky_relu(h + _gcn(h, row, col, dis, W2, b2), 0.01)
    h = _batchnorm(h, bn2_g, bn2_b)
    h = _gat(h, row, col, Wg, att_src, att_dst, bg)
    h = _layernorm(h, ln2_g, ln2_b)
    h = jax.nn.leaky_relu(_gcn(h, row, col, dis, W3, b3), 0.01)
    h = _batchnorm(h, bn3_g, bn3_b)

    w_cat = jnp.concatenate(
        [fc1_w, fc2_w, jnp.zeros((256, 111), jnp.float32)], axis=1)
    b_cat = jnp.concatenate([fc1_b, fc2_b, jnp.zeros((111,), jnp.float32)])
    out = _mm(h, w_cat, b_cat, act=True)
    return (out[:, 0:1], out[:, 1:17])
